# baseline (speedup 1.0000x reference)
import jax
import jax.numpy as jnp
from jax import lax
from jax.experimental import pallas as pl
from jax.experimental.pallas import tpu as pltpu

B = 128
C = 100000
D = 128

BC = 16384
NB = (C + BC - 1) // BC
TAIL = 2048


def _eye():
    return (lax.broadcasted_iota(jnp.int32, (B, B), 0) ==
            lax.broadcasted_iota(jnp.int32, (B, B), 1)).astype(jnp.float32)


def _fused_body(ytc_ref, ytr_ref, x_ref, w_ref, o_ref,
                m_s, s_s, b_s, t_s, wjt_s, cw_s):
    i = pl.program_id(0)

    @pl.when(i == 0)
    def _init():
        m_s[...] = jnp.full((B, 1), -jnp.inf, jnp.float32)
        s_s[...] = jnp.zeros((B, 1), jnp.float32)
        b_s[...] = jnp.zeros((B, 1), jnp.int32)
        t_s[...] = jnp.zeros((B, 1), jnp.float32)
        wjt_s[...] = jnp.zeros((B, D), jnp.float32)

    def _stream_update(xm, wb, gcol_r, gcol_c):
        m_old = m_s[...]
        bm = jnp.max(xm, axis=1, keepdims=True)
        m_new = jnp.maximum(m_old, bm)
        s_s[...] = s_s[...] * jnp.exp(m_old - m_new) + jnp.sum(
            jnp.exp(xm - m_new), axis=1, keepdims=True)
        m_s[...] = m_new
        b_s[...] = jnp.where(bm > m_old, i, b_s[...])

        eqt = gcol_r == ytc_ref[...]
        t_s[...] = t_s[...] + jnp.sum(
            jnp.where(eqt, xm, 0.0), axis=1, keepdims=True)
        wjt_s[...] = wjt_s[...] + lax.dot_general(
            eqt.astype(jnp.bfloat16), wb, (((1,), (1,)), ((), ())),
            preferred_element_type=jnp.float32)

        cand = jnp.min(jnp.where(xm == bm, gcol_r, jnp.int32(2**30)),
                       axis=1, keepdims=True)
        cand_row = lax.dot_general(
            cand.astype(jnp.float32), _eye(), (((0,), (0,)), ((), ())),
            preferred_element_type=jnp.float32)
        ohp = (gcol_c.astype(jnp.float32) == cand_row).astype(jnp.bfloat16)
        cw_s[pl.ds(i * D, D), :] = lax.dot_general(
            wb, ohp, (((1,), (0,)), ((), ())),
            preferred_element_type=jnp.float32)

    @pl.when(i < NB - 1)
    def _stream_fast():
        gcol_r = i * BC + lax.broadcasted_iota(jnp.int32, (1, BC), 1)
        gcol_c = i * BC + lax.broadcasted_iota(jnp.int32, (BC, 1), 0)
        _stream_update(x_ref[...], w_ref[...].astype(jnp.bfloat16),
                       gcol_r, gcol_c)

    @pl.when(i == NB - 1)
    def _stream_tail_and_finish():
        gcol_r = i * BC + lax.broadcasted_iota(jnp.int32, (1, TAIL), 1)
        gcol_c = i * BC + lax.broadcasted_iota(jnp.int32, (TAIL, 1), 0)
        valid = gcol_r < C
        xm = jnp.where(valid, x_ref[:, 0:TAIL], -jnp.inf)
        wb = jnp.where(valid, w_ref[:, 0:TAIL], 0.0).astype(jnp.bfloat16)
        _stream_update(xm, wb, gcol_r, gcol_c)

        kstar_row = lax.dot_general(
            b_s[...].astype(jnp.float32), _eye(), (((0,), (0,)), ((), ())),
            preferred_element_type=jnp.float32)
        wi = jnp.zeros((D, B), jnp.float32)
        for k in range(NB):
            sel = (kstar_row == float(k)).astype(jnp.float32)
            wi = wi + cw_s[k * D:(k + 1) * D, :] * sel

        mm = lax.dot_general(wjt_s[...], wi, (((1,), (0,)), ((), ())),
                             preferred_element_type=jnp.float32)
        mcol = jnp.sum(mm * _eye(), axis=1, keepdims=True)
        m = m_s[...]
        t = t_s[...]
        zz = s_s[...] - jnp.exp(t - m) + jnp.exp(t - mcol - m)
        lossv = m + jnp.log(zz) - t + mcol
        o_ref[...] = jnp.sum(lossv, axis=0, keepdims=True) * (1.0 / B)


_fused = pl.pallas_call(
    _fused_body,
    grid=(NB,),
    in_specs=[
        pl.BlockSpec((B, 1), lambda i: (0, 0)),
        pl.BlockSpec((1, B), lambda i: (0, 0)),
        pl.BlockSpec((B, BC), lambda i: (0, i)),
        pl.BlockSpec((D, BC), lambda i: (0, i)),
    ],
    out_specs=pl.BlockSpec((1, 1), lambda i: (0, 0)),
    out_shape=jax.ShapeDtypeStruct((1, 1), jnp.float32),
    scratch_shapes=[
        pltpu.VMEM((B, 1), jnp.float32),
        pltpu.VMEM((B, 1), jnp.float32),
        pltpu.VMEM((B, 1), jnp.int32),
        pltpu.VMEM((B, 1), jnp.float32),
        pltpu.VMEM((B, D), jnp.float32),
        pltpu.VMEM((NB * D, B), jnp.float32),
    ],
    compiler_params=pltpu.CompilerParams(
        dimension_semantics=("arbitrary",)),
)


@jax.jit
def kernel(y_pred, y_true, W):
    y_true = y_true.astype(jnp.int32)
    ytc = y_true.reshape(B, 1)
    ytr = y_true.astype(jnp.float32).reshape(1, B)
    loss = _fused(ytc, ytr, y_pred, W)
    return loss.reshape(())


# speedup vs baseline: 1.0017x; 1.0017x over previous
import jax
import jax.numpy as jnp
from jax import lax
from jax.experimental import pallas as pl
from jax.experimental.pallas import tpu as pltpu

B = 128
C = 100000
D = 128

BC = 16384
NB = (C + BC - 1) // BC
TAIL = 2048


def _eye():
    return (lax.broadcasted_iota(jnp.int32, (B, B), 0) ==
            lax.broadcasted_iota(jnp.int32, (B, B), 1)).astype(jnp.float32)


def _fused_body(ytc_ref, ytr_ref, x_ref, w_ref, o_ref,
                m_s, s_s, b_s, t_s, wjt_s, cw_s, xs_s):
    i = pl.program_id(0)

    @pl.when(i == 0)
    def _init():
        m_s[...] = jnp.full((B, 1), -jnp.inf, jnp.float32)
        s_s[...] = jnp.zeros((B, 1), jnp.float32)
        b_s[...] = jnp.zeros((B, 1), jnp.int32)
        t_s[...] = jnp.zeros((B, 1), jnp.float32)
        wjt_s[...] = jnp.zeros((B, D), jnp.float32)

    def _stream_update(xm, wb, gcol_r, gcol_c):
        m_old = m_s[...]
        bm = jnp.max(xm, axis=1, keepdims=True)
        m_new = jnp.maximum(m_old, bm)
        s_s[...] = s_s[...] * jnp.exp(m_old - m_new) + jnp.sum(
            jnp.exp(xm - m_new), axis=1, keepdims=True)
        m_s[...] = m_new
        b_s[...] = jnp.where(bm > m_old, i, b_s[...])

        eqt = gcol_r == ytc_ref[...]
        t_s[...] = t_s[...] + jnp.sum(
            jnp.where(eqt, xm, 0.0), axis=1, keepdims=True)
        wjt_s[...] = wjt_s[...] + lax.dot_general(
            eqt.astype(jnp.bfloat16), wb, (((1,), (1,)), ((), ())),
            preferred_element_type=jnp.float32)

        cand = jnp.min(jnp.where(xm == bm, gcol_r, jnp.int32(2**30)),
                       axis=1, keepdims=True)
        cand_row = lax.dot_general(
            cand.astype(jnp.float32), _eye(), (((0,), (0,)), ((), ())),
            preferred_element_type=jnp.float32)
        ohp = (gcol_c.astype(jnp.float32) == cand_row).astype(jnp.bfloat16)
        cw_s[pl.ds(i * D, D), :] = lax.dot_general(
            wb, ohp, (((1,), (0,)), ((), ())),
            preferred_element_type=jnp.float32)

    @pl.when(i < NB - 1)
    def _stream_fast():
        xs_s[...] = x_ref[...]
        wb = w_ref[...].astype(jnp.bfloat16)
        gcol_r = i * BC + lax.broadcasted_iota(jnp.int32, (1, BC), 1)
        gcol_c = i * BC + lax.broadcasted_iota(jnp.int32, (BC, 1), 0)
        _stream_update(xs_s[...], wb, gcol_r, gcol_c)

    @pl.when(i == NB - 1)
    def _stream_tail_and_finish():
        gcol_r = i * BC + lax.broadcasted_iota(jnp.int32, (1, TAIL), 1)
        gcol_c = i * BC + lax.broadcasted_iota(jnp.int32, (TAIL, 1), 0)
        valid = gcol_r < C
        xm = jnp.where(valid, x_ref[:, 0:TAIL], -jnp.inf)
        wb = jnp.where(valid, w_ref[:, 0:TAIL], 0.0).astype(jnp.bfloat16)
        _stream_update(xm, wb, gcol_r, gcol_c)

        kstar_row = lax.dot_general(
            b_s[...].astype(jnp.float32), _eye(), (((0,), (0,)), ((), ())),
            preferred_element_type=jnp.float32)
        wi = jnp.zeros((D, B), jnp.float32)
        for k in range(NB):
            sel = (kstar_row == float(k)).astype(jnp.float32)
            wi = wi + cw_s[k * D:(k + 1) * D, :] * sel

        mm = lax.dot_general(wjt_s[...], wi, (((1,), (0,)), ((), ())),
                             preferred_element_type=jnp.float32)
        mcol = jnp.sum(mm * _eye(), axis=1, keepdims=True)
        m = m_s[...]
        t = t_s[...]
        zz = s_s[...] - jnp.exp(t - m) + jnp.exp(t - mcol - m)
        lossv = m + jnp.log(zz) - t + mcol
        o_ref[...] = jnp.sum(lossv, axis=0, keepdims=True) * (1.0 / B)


_fused = pl.pallas_call(
    _fused_body,
    grid=(NB,),
    in_specs=[
        pl.BlockSpec((B, 1), lambda i: (0, 0)),
        pl.BlockSpec((1, B), lambda i: (0, 0)),
        pl.BlockSpec((B, BC), lambda i: (0, i)),
        pl.BlockSpec((D, BC), lambda i: (0, i)),
    ],
    out_specs=pl.BlockSpec((1, 1), lambda i: (0, 0)),
    out_shape=jax.ShapeDtypeStruct((1, 1), jnp.float32),
    scratch_shapes=[
        pltpu.VMEM((B, 1), jnp.float32),
        pltpu.VMEM((B, 1), jnp.float32),
        pltpu.VMEM((B, 1), jnp.int32),
        pltpu.VMEM((B, 1), jnp.float32),
        pltpu.VMEM((B, D), jnp.float32),
        pltpu.VMEM((NB * D, B), jnp.float32),
        pltpu.VMEM((B, BC), jnp.float32),
    ],
    compiler_params=pltpu.CompilerParams(
        dimension_semantics=("arbitrary",)),
)


@jax.jit
def kernel(y_pred, y_true, W):
    y_true = y_true.astype(jnp.int32)
    ytc = y_true.reshape(B, 1)
    ytr = y_true.astype(jnp.float32).reshape(1, B)
    loss = _fused(ytc, ytr, y_pred, W)
    return loss.reshape(())
